# baseline (speedup 1.0000x reference)
import functools

import jax
import jax.numpy as jnp
from jax import lax
from jax.experimental import pallas as pl
from jax.experimental.pallas import tpu as pltpu
from jax.experimental.pallas import tpu_sc as plsc

B = 16384
L = 50
D = 128
NV = D // 16
G = 2
GW = G * L
CE = 16
NG = CE // G
UN = 10


def _make_pool():
  info = plsc.get_sparse_core_info()
  nc, ns = info.num_cores, info.num_subcores
  nw = nc * ns
  e_total = 2 * B
  epw = e_total // nw
  nch = epw // CE
  npair = nch // 2
  grows_total = e_total * L // GW
  gpw = epw // G
  mesh = plsc.VectorSubcoreMesh(core_axis_name="c", subcore_axis_name="s")

  @functools.partial(
      pl.kernel,
      out_type=jax.ShapeDtypeStruct((e_total, D), jnp.float32),
      mesh=mesh,
      scratch_types=[
          pltpu.VMEM((NG, GW), jnp.int32),
          pltpu.VMEM((NG, GW), jnp.int32),
          pltpu.VMEM((GW, D), jnp.float32),
          pltpu.VMEM((GW, D), jnp.float32),
          pltpu.VMEM((GW, D), jnp.float32),
          pltpu.VMEM((GW, D), jnp.float32),
          pltpu.VMEM((CE, D), jnp.float32),
          pltpu.VMEM((CE, D), jnp.float32),
          pltpu.SemaphoreType.DMA,
          pltpu.SemaphoreType.DMA,
          pltpu.SemaphoreType.DMA,
          pltpu.SemaphoreType.DMA,
          pltpu.SemaphoreType.DMA,
          pltpu.SemaphoreType.DMA,
          pltpu.SemaphoreType.DMA,
          pltpu.SemaphoreType.DMA,
      ],
  )
  def pool(embed_hbm, idx_hbm, out_hbm, idx0, idx1, rows0, rows1, rows2,
           rows3, acc0, acc1, gsem0, gsem1, gsem2, gsem3, isem0, isem1,
           osem0, osem1):
    wid = lax.axis_index("s") * nc + lax.axis_index("c")
    grow0 = wid * gpw
    rows = (rows0, rows1, rows2, rows3)
    gsems = (gsem0, gsem1, gsem2, gsem3)
    rd = len(rows)
    ahead = rd - 1

    def fire_idx(c, buf, sem):
      row = jnp.minimum(grow0 + c * NG, grows_total - NG)
      return pltpu.async_copy(idx_hbm.at[pl.ds(row, NG)], buf, sem)

    def fire_gather(idxb, p, par):
      return pltpu.async_copy(embed_hbm.at[idxb.at[p]], rows[par],
                              gsems[par])

    def zero_accs():
      return tuple(jnp.zeros((16,), jnp.float32) for _ in range(G * NV))

    pltpu.sync_copy(idx_hbm.at[pl.ds(grow0, NG)], idx0)
    fire_idx(1, idx1, isem1)
    for p in range(ahead):
      fire_gather(idx0, p, p % rd)

    def do_chunk(c, idxb, acc, idx_other, isem_other, isem_self,
                 osem_self):
      cps = {}
      for p in range(NG):
        nf = p + ahead
        if nf < NG:
          cps[nf] = fire_gather(idxb, nf, nf % rd)
        else:
          if nf == NG:
            pltpu.make_async_copy(idx_hbm.at[pl.ds(0, NG)], idx_other,
                                  isem_other).wait()
          fire_gather(idx_other, nf - NG, (nf - NG) % rd)
          if p == NG - 1:
            fire_idx(c + 2, idxb, isem_self)
        if p < ahead:
          pltpu.make_async_copy(embed_hbm.at[idxb.at[p]], rows[p % rd],
                                gsems[p % rd]).wait()
        else:
          cps[p].wait()
        buf = rows[p % rd]

        def rbody(r, accs, _buf=buf):
          accs = list(accs)
          for u in range(UN):
            for e in range(G):
              rr = e * L + r * UN + u
              for d in range(NV):
                accs[e * NV + d] = (accs[e * NV + d]
                                    + _buf[rr, pl.ds(16 * d, 16)])
          return tuple(accs)

        accs = lax.fori_loop(0, L // UN, rbody, zero_accs())
        for e in range(G):
          for d in range(NV):
            acc[p * G + e, pl.ds(16 * d, 16)] = accs[e * NV + d]
      return pltpu.async_copy(
          acc, out_hbm.at[pl.ds(wid * epw + c * CE, CE)], osem_self)

    def pair_body(t, carry):
      oc0 = do_chunk(2 * t, idx0, acc0, idx1, isem1, isem0, osem0)
      oc1 = do_chunk(2 * t + 1, idx1, acc1, idx0, isem0, isem1, osem1)
      oc0.wait()
      oc1.wait()
      return carry

    lax.fori_loop(0, npair, pair_body, 0)
    for p in range(ahead):
      pltpu.make_async_copy(embed_hbm.at[idx0.at[p]], rows[p % rd],
                            gsems[p % rd]).wait()
    pltpu.make_async_copy(idx_hbm.at[pl.ds(0, NG)], idx1, isem1).wait()

  return pool


NB = 16
BS = B // NB


def _dense_body(p1, p2, x3, b1, w1, f1, b2, w2, f2, e2, o):
  h1 = jnp.tanh(p1[...] + b1[...])
  h1 = jnp.tanh(lax.dot_general(h1, w1[...], (((1,), (0,)), ((), ())),
                                preferred_element_type=jnp.float32) + f1[...])
  h2 = jnp.tanh(p2[...] + b2[...])
  h2 = jnp.tanh(lax.dot_general(h2, w2[...], (((1,), (0,)), ((), ())),
                                preferred_element_type=jnp.float32) + f2[...])
  x12 = jnp.sum(h1 * h2, axis=1)
  cols = lax.broadcasted_iota(jnp.int32, (BS, D), 1)
  h3 = jnp.sum(jnp.where(cols == x3[...], e2[...], 0.0), axis=1)
  o[...] = jax.nn.sigmoid(x12 + h3)


def _dense(pooled, x3i, b1, w1, f1, b2, w2, f2, e2):
  full = pl.BlockSpec((1, D), lambda i: (0, 0))
  return pl.pallas_call(
      _dense_body,
      grid=(NB,),
      in_specs=[
          pl.BlockSpec((BS, D), lambda i: (i, 0)),
          pl.BlockSpec((BS, D), lambda i: (i + NB, 0)),
          pl.BlockSpec((BS, 1), lambda i: (i, 0)),
          full,
          pl.BlockSpec((D, D), lambda i: (0, 0)),
          full,
          full,
          pl.BlockSpec((D, D), lambda i: (0, 0)),
          full,
          full,
      ],
      out_specs=pl.BlockSpec((BS,), lambda i: (i,)),
      out_shape=jax.ShapeDtypeStruct((B,), jnp.float32),
  )(pooled, pooled, x3i, b1, w1, f1, b2, w2, f2, e2)


def kernel(x1, x2, x3, embed, t1_bias1, t1_fc2_w, t1_fc2_b,
           t2_bias1, t2_fc2_w, t2_fc2_b, embed2):
  idx = jnp.concatenate([x1, x2], axis=0).astype(jnp.int32)
  idx = idx.reshape(2 * B * L // GW, GW)
  pooled = _make_pool()(embed.astype(jnp.float32), idx)

  b1 = t1_bias1.reshape(1, D)
  b2 = t2_bias1.reshape(1, D)
  w1 = jnp.pad(t1_fc2_w, ((0, D - t1_fc2_w.shape[0]), (0, 0))).T
  w2 = jnp.pad(t2_fc2_w, ((0, D - t2_fc2_w.shape[0]), (0, 0))).T
  f1 = jnp.pad(t1_fc2_b, (0, D - t1_fc2_b.shape[0])).reshape(1, D)
  f2 = jnp.pad(t2_fc2_b, (0, D - t2_fc2_b.shape[0])).reshape(1, D)
  e2 = jnp.pad(embed2[:, 0], (0, D - embed2.shape[0])).reshape(1, D)
  x3i = x3.astype(jnp.int32)

  return _dense(pooled, x3i, b1, w1, f1, b2, w2, f2, e2)


# speedup vs baseline: 1.5691x; 1.5691x over previous
import functools

import jax
import jax.numpy as jnp
from jax import lax
from jax.experimental import pallas as pl
from jax.experimental.pallas import tpu as pltpu
from jax.experimental.pallas import tpu_sc as plsc

B = 16384
L = 50
D = 128
NV = D // 16
G = 2
GW = G * L
CE = 16
NG = CE // G
UN = 5


def _make_pool():
  info = plsc.get_sparse_core_info()
  nc, ns = info.num_cores, info.num_subcores
  nw = nc * ns
  e_total = 2 * B
  epw = e_total // nw
  nch = epw // CE
  npair = nch // 2
  grows_total = e_total * L // GW
  gpw = epw // G
  mesh = plsc.VectorSubcoreMesh(core_axis_name="c", subcore_axis_name="s")

  @functools.partial(
      pl.kernel,
      out_type=jax.ShapeDtypeStruct((e_total, D), jnp.float32),
      mesh=mesh,
      scratch_types=[
          pltpu.VMEM((NG, GW), jnp.int32),
          pltpu.VMEM((NG, GW), jnp.int32),
          pltpu.VMEM((GW, D), jnp.float32),
          pltpu.VMEM((GW, D), jnp.float32),
          pltpu.VMEM((GW, D), jnp.float32),
          pltpu.VMEM((GW, D), jnp.float32),
          pltpu.VMEM((GW, D), jnp.float32),
          pltpu.VMEM((GW, D), jnp.float32),
          pltpu.VMEM((GW, D), jnp.float32),
          pltpu.VMEM((GW, D), jnp.float32),
          pltpu.VMEM((CE, D), jnp.float32),
          pltpu.VMEM((CE, D), jnp.float32),
          pltpu.SemaphoreType.DMA,
          pltpu.SemaphoreType.DMA,
          pltpu.SemaphoreType.DMA,
          pltpu.SemaphoreType.DMA,
          pltpu.SemaphoreType.DMA,
          pltpu.SemaphoreType.DMA,
          pltpu.SemaphoreType.DMA,
          pltpu.SemaphoreType.DMA,
          pltpu.SemaphoreType.DMA,
          pltpu.SemaphoreType.DMA,
          pltpu.SemaphoreType.DMA,
          pltpu.SemaphoreType.DMA,
      ],
  )
  def pool(embed_hbm, idx_hbm, out_hbm, idx0, idx1, rows0, rows1, rows2,
           rows3, rows4, rows5, rows6, rows7, acc0, acc1, gsem0, gsem1,
           gsem2, gsem3, gsem4, gsem5, gsem6, gsem7, isem0, isem1,
           osem0, osem1):
    wid = lax.axis_index("s") * nc + lax.axis_index("c")
    grow0 = wid * gpw
    rows = (rows0, rows1, rows2, rows3, rows4, rows5, rows6, rows7)
    gsems = (gsem0, gsem1, gsem2, gsem3, gsem4, gsem5, gsem6, gsem7)
    rd = len(rows)
    ahead = rd - 1

    def fire_idx(c, buf, sem):
      row = jnp.minimum(grow0 + c * NG, grows_total - NG)
      return pltpu.async_copy(idx_hbm.at[pl.ds(row, NG)], buf, sem)

    def fire_gather(idxb, p, par):
      return pltpu.async_copy(embed_hbm.at[idxb.at[p]], rows[par],
                              gsems[par])

    def zero_accs():
      return tuple(jnp.zeros((16,), jnp.float32) for _ in range(G * NV))

    pltpu.sync_copy(idx_hbm.at[pl.ds(grow0, NG)], idx0)
    fire_idx(1, idx1, isem1)
    for p in range(ahead):
      fire_gather(idx0, p, p % rd)

    def do_chunk(c, idxb, acc, idx_other, isem_other, isem_self,
                 osem_self):
      cps = {}
      for p in range(NG):
        nf = p + ahead
        if nf < NG:
          cps[nf] = fire_gather(idxb, nf, nf % rd)
        else:
          if nf == NG:
            pltpu.make_async_copy(idx_hbm.at[pl.ds(0, NG)], idx_other,
                                  isem_other).wait()
          fire_gather(idx_other, nf - NG, (nf - NG) % rd)
          if p == NG - 1:
            fire_idx(c + 2, idxb, isem_self)
        if p < ahead:
          pltpu.make_async_copy(embed_hbm.at[idxb.at[p]], rows[p % rd],
                                gsems[p % rd]).wait()
        else:
          cps[p].wait()
        buf = rows[p % rd]

        def rbody(r, accs, _buf=buf):
          accs = list(accs)
          for u in range(UN):
            for e in range(G):
              rr = e * L + r * UN + u
              for d in range(NV):
                accs[e * NV + d] = (accs[e * NV + d]
                                    + _buf[rr, pl.ds(16 * d, 16)])
          return tuple(accs)

        accs = lax.fori_loop(0, L // UN, rbody, zero_accs())
        for e in range(G):
          for d in range(NV):
            acc[p * G + e, pl.ds(16 * d, 16)] = accs[e * NV + d]
      return pltpu.async_copy(
          acc, out_hbm.at[pl.ds(wid * epw + c * CE, CE)], osem_self)

    def pair_body(t, carry):
      oc0 = do_chunk(2 * t, idx0, acc0, idx1, isem1, isem0, osem0)
      oc1 = do_chunk(2 * t + 1, idx1, acc1, idx0, isem0, isem1, osem1)
      oc0.wait()
      oc1.wait()
      return carry

    lax.fori_loop(0, npair, pair_body, 0)
    for p in range(ahead):
      pltpu.make_async_copy(embed_hbm.at[idx0.at[p]], rows[p % rd],
                            gsems[p % rd]).wait()
    pltpu.make_async_copy(idx_hbm.at[pl.ds(0, NG)], idx1, isem1).wait()

  return pool


NB = 16
BS = B // NB


def _dense_body(p1, p2, x3, b1, w1, f1, b2, w2, f2, e2, o):
  h1 = jnp.tanh(p1[...] + b1[...])
  h1 = jnp.tanh(lax.dot_general(h1, w1[...], (((1,), (0,)), ((), ())),
                                preferred_element_type=jnp.float32) + f1[...])
  h2 = jnp.tanh(p2[...] + b2[...])
  h2 = jnp.tanh(lax.dot_general(h2, w2[...], (((1,), (0,)), ((), ())),
                                preferred_element_type=jnp.float32) + f2[...])
  x12 = jnp.sum(h1 * h2, axis=1)
  cols = lax.broadcasted_iota(jnp.int32, (BS, D), 1)
  h3 = jnp.sum(jnp.where(cols == x3[...], e2[...], 0.0), axis=1)
  o[...] = jax.nn.sigmoid(x12 + h3)


def _dense(pooled, x3i, b1, w1, f1, b2, w2, f2, e2):
  full = pl.BlockSpec((1, D), lambda i: (0, 0))
  return pl.pallas_call(
      _dense_body,
      grid=(NB,),
      in_specs=[
          pl.BlockSpec((BS, D), lambda i: (i, 0)),
          pl.BlockSpec((BS, D), lambda i: (i + NB, 0)),
          pl.BlockSpec((BS, 1), lambda i: (i, 0)),
          full,
          pl.BlockSpec((D, D), lambda i: (0, 0)),
          full,
          full,
          pl.BlockSpec((D, D), lambda i: (0, 0)),
          full,
          full,
      ],
      out_specs=pl.BlockSpec((BS,), lambda i: (i,)),
      out_shape=jax.ShapeDtypeStruct((B,), jnp.float32),
  )(pooled, pooled, x3i, b1, w1, f1, b2, w2, f2, e2)


def kernel(x1, x2, x3, embed, t1_bias1, t1_fc2_w, t1_fc2_b,
           t2_bias1, t2_fc2_w, t2_fc2_b, embed2):
  idx = jnp.concatenate([x1, x2], axis=0).astype(jnp.int32)
  idx = idx.reshape(2 * B * L // GW, GW)
  pooled = _make_pool()(embed.astype(jnp.float32), idx)

  b1 = t1_bias1.reshape(1, D)
  b2 = t2_bias1.reshape(1, D)
  w1 = jnp.pad(t1_fc2_w, ((0, D - t1_fc2_w.shape[0]), (0, 0))).T
  w2 = jnp.pad(t2_fc2_w, ((0, D - t2_fc2_w.shape[0]), (0, 0))).T
  f1 = jnp.pad(t1_fc2_b, (0, D - t1_fc2_b.shape[0])).reshape(1, D)
  f2 = jnp.pad(t2_fc2_b, (0, D - t2_fc2_b.shape[0])).reshape(1, D)
  e2 = jnp.pad(embed2[:, 0], (0, D - embed2.shape[0])).reshape(1, D)
  x3i = x3.astype(jnp.int32)

  return _dense(pooled, x3i, b1, w1, f1, b2, w2, f2, e2)
